# baseline (speedup 1.0000x reference)
import functools
import math

import jax
import jax.numpy as jnp
from jax.experimental import pallas as pl
from jax.experimental.pallas import tpu as pltpu

IN_F = 4096
OUT_F = 4096
RANK = 8
ALPHA = 32
NUM_EXPERTS = 64
TOP_K = 16
ROUTER_DIM = 16
BOTTLENECK = NUM_EXPERTS * RANK
SCALING = ALPHA / TOP_K

BT = 512

NEG_INF = float("-inf")


def _body(n_blocks, x_ref, aw_ref, bw_ref, wrd_ref, wru_ref, out_ref, zg_ref):
    i = pl.program_id(0)

    @pl.when(i < n_blocks)
    def _gate_phase():
        x = x_ref[...]

        s_lo = jnp.dot(x, wrd_ref[...], preferred_element_type=jnp.float32)
        scores = jnp.dot(s_lo, wru_ref[...], preferred_element_type=jnp.float32)

        thr = jnp.max(scores, axis=-1, keepdims=True)
        for _ in range(TOP_K - 1):
            below = jnp.where(scores < thr, scores, NEG_INF)
            thr = jnp.max(below, axis=-1, keepdims=True)
        rowmax = jnp.max(scores, axis=-1, keepdims=True)
        p = jnp.where(scores >= thr, jnp.exp(scores - rowmax), 0.0)
        gate = p * (SCALING / jnp.sum(p, axis=-1, keepdims=True))

        z = jnp.dot(x.astype(jnp.bfloat16), aw_ref[...],
                    preferred_element_type=jnp.float32)

        r = jax.lax.broadcasted_iota(jnp.int32, (NUM_EXPERTS, BOTTLENECK), 0)
        c = jax.lax.broadcasted_iota(jnp.int32, (NUM_EXPERTS, BOTTLENECK), 1)
        expand = (c // RANK == r).astype(jnp.float32)
        gate_exp = jnp.dot(gate, expand, preferred_element_type=jnp.float32)

        zg_ref[i % 2] = (z * gate_exp).astype(jnp.bfloat16)

    @pl.when(i > 0)
    def _out_phase():
        out_ref[...] = jnp.dot(zg_ref[(i - 1) % 2], bw_ref[...],
                               preferred_element_type=jnp.float32)


@jax.jit
def kernel(x, A_w, B_w, Wr_down, Wr_up):
    orig_shape = x.shape
    T = math.prod(orig_shape[:-1])
    x2 = x.reshape(T, IN_F)
    n = T // BT
    out = pl.pallas_call(
        functools.partial(_body, n),
        grid=(n + 1,),
        in_specs=[
            pl.BlockSpec((BT, IN_F), lambda i: (jnp.minimum(i, n - 1), 0)),
            pl.BlockSpec((IN_F, BOTTLENECK), lambda i: (0, 0)),
            pl.BlockSpec((BOTTLENECK, OUT_F), lambda i: (0, 0)),
            pl.BlockSpec((IN_F, ROUTER_DIM), lambda i: (0, 0)),
            pl.BlockSpec((ROUTER_DIM, NUM_EXPERTS), lambda i: (0, 0)),
        ],
        out_specs=pl.BlockSpec((BT, OUT_F), lambda i: (jnp.maximum(i - 1, 0), 0)),
        out_shape=jax.ShapeDtypeStruct((T, OUT_F), jnp.float32),
        scratch_shapes=[pltpu.VMEM((2, BT, BOTTLENECK), jnp.bfloat16)],
    )(x2, A_w.astype(jnp.bfloat16), B_w.astype(jnp.bfloat16), Wr_down, Wr_up)
    return out.reshape(*orig_shape[:-1], OUT_F)


# speedup vs baseline: 1.0776x; 1.0776x over previous
import functools
import math

import jax
import jax.numpy as jnp
from jax.experimental import pallas as pl
from jax.experimental.pallas import tpu as pltpu

IN_F = 4096
OUT_F = 4096
RANK = 8
ALPHA = 32
NUM_EXPERTS = 64
TOP_K = 16
ROUTER_DIM = 16
BOTTLENECK = NUM_EXPERTS * RANK
SCALING = ALPHA / TOP_K

BT = 512

NEG_INF = float("-inf")


def _body(n_blocks, x_ref, aw_ref, bw_ref, wr_ref, out_ref, zg_ref):
    i = pl.program_id(0)

    @pl.when(i < n_blocks)
    def _gate_phase():
        x = x_ref[...]

        scores = jnp.dot(x, wr_ref[...], preferred_element_type=jnp.float32)

        thr = jnp.max(scores, axis=-1, keepdims=True)
        for _ in range(TOP_K - 1):
            below = jnp.where(scores < thr, scores, NEG_INF)
            thr = jnp.max(below, axis=-1, keepdims=True)
        rowmax = jnp.max(scores, axis=-1, keepdims=True)
        p = jnp.where(scores >= thr, jnp.exp(scores - rowmax), 0.0)
        gate = p * (SCALING / jnp.sum(p, axis=-1, keepdims=True))

        z = jnp.dot(x.astype(jnp.bfloat16), aw_ref[...].astype(jnp.bfloat16),
                    preferred_element_type=jnp.float32)

        r = jax.lax.broadcasted_iota(jnp.int32, (NUM_EXPERTS, BOTTLENECK), 0)
        c = jax.lax.broadcasted_iota(jnp.int32, (NUM_EXPERTS, BOTTLENECK), 1)
        expand = (c // RANK == r).astype(jnp.bfloat16)
        gate_exp = jnp.dot(gate.astype(jnp.bfloat16), expand,
                           preferred_element_type=jnp.float32)

        zg_ref[i % 2] = (z * gate_exp).astype(jnp.bfloat16)

    @pl.when(i > 0)
    def _out_phase():
        out_ref[...] = jnp.dot(zg_ref[(i - 1) % 2],
                               bw_ref[...].astype(jnp.bfloat16),
                               preferred_element_type=jnp.float32)


@jax.jit
def kernel(x, A_w, B_w, Wr_down, Wr_up):
    orig_shape = x.shape
    T = math.prod(orig_shape[:-1])
    x2 = x.reshape(T, IN_F)
    n = T // BT
    out = pl.pallas_call(
        functools.partial(_body, n),
        grid=(n + 1,),
        in_specs=[
            pl.BlockSpec((BT, IN_F), lambda i: (jnp.minimum(i, n - 1), 0)),
            pl.BlockSpec((IN_F, BOTTLENECK), lambda i: (0, 0)),
            pl.BlockSpec((BOTTLENECK, OUT_F), lambda i: (0, 0)),
            pl.BlockSpec((IN_F, NUM_EXPERTS), lambda i: (0, 0)),
        ],
        out_specs=pl.BlockSpec((BT, OUT_F), lambda i: (jnp.maximum(i - 1, 0), 0)),
        out_shape=jax.ShapeDtypeStruct((T, OUT_F), jnp.float32),
        scratch_shapes=[pltpu.VMEM((2, BT, BOTTLENECK), jnp.bfloat16)],
    )(x2, A_w, B_w, Wr_down @ Wr_up)
    return out.reshape(*orig_shape[:-1], OUT_F)
